# Initial kernel scaffold; baseline (speedup 1.0000x reference)
import jax
import jax.numpy as jnp
from jax.experimental import pallas as pl


def kernel(x, edge_index, W1, b1, W2, b2, Wr, br):
    raise NotImplementedError("write your pallas kernel here")


# speedup vs baseline: 13.3886x; 13.3886x over previous
import functools

import jax
import jax.numpy as jnp
from jax import lax
from jax.experimental import pallas as pl
from jax.experimental.pallas import tpu as pltpu
from jax.experimental.pallas import tpu_sc as plsc

N = 10000
D = 128
NC, NS = 2, 16
NW = NC * NS
B = 128
N_PAD = 10240
ROWS_PER_TILE = N_PAD // NS
R = 2048

_MESH = plsc.VectorSubcoreMesh(
    core_axis_name="c", subcore_axis_name="s", num_cores=NC, num_subcores=NS)


def _make_deg_kernel(k):
    @functools.partial(
        pl.kernel,
        out_type=jax.ShapeDtypeStruct((NC, N_PAD), jnp.float32),
        mesh=_MESH,
        scratch_types=[
            pltpu.VMEM((k, B), jnp.int32),
            pltpu.VMEM((ROWS_PER_TILE + B,), jnp.float32),
            pltpu.VMEM_SHARED((N_PAD,), jnp.float32),
        ],
    )
    def deg_kernel(dst_hbm, const_hbm, out_hbm, dst_v, const_v, acc):
        c = lax.axis_index("c")
        s = lax.axis_index("s")
        wid = c * NS + s
        pltpu.sync_copy(dst_hbm.at[wid], dst_v)
        pltpu.sync_copy(const_hbm, const_v)
        pltpu.sync_copy(const_v.at[pl.ds(0, ROWS_PER_TILE)],
                        acc.at[pl.ds(s * ROWS_PER_TILE, ROWS_PER_TILE)])
        plsc.subcore_barrier()

        def body(j, carry):
            pltpu.sync_copy(const_v.at[pl.ds(ROWS_PER_TILE, B)],
                            acc.at[dst_v.at[j]], add=True)
            return carry

        lax.fori_loop(0, k, body, 0)
        plsc.subcore_barrier()
        pltpu.sync_copy(acc.at[pl.ds(s * ROWS_PER_TILE, ROWS_PER_TILE)],
                        out_hbm.at[c, pl.ds(s * ROWS_PER_TILE, ROWS_PER_TILE)])

    return deg_kernel


def _make_pass_kernel(k):
    @functools.partial(
        pl.kernel,
        out_type=jax.ShapeDtypeStruct((NC, N_PAD, D), jnp.float32),
        mesh=_MESH,
        scratch_types=[
            pltpu.VMEM((k, B), jnp.int32),
            pltpu.VMEM((k, B), jnp.int32),
            pltpu.VMEM((B, D), jnp.float32),
            pltpu.VMEM_SHARED((N_PAD, D), jnp.float32),
            pltpu.SemaphoreType.DMA,
        ],
    )
    def pass_kernel(y_hbm, src_hbm, dst_hbm, zeros_hbm, out_hbm,
                    src_v, dst_v, rows_v, acc, sem):
        c = lax.axis_index("c")
        s = lax.axis_index("s")
        wid = c * NS + s
        pltpu.sync_copy(src_hbm.at[wid], src_v)
        pltpu.sync_copy(dst_hbm.at[wid], dst_v)
        pltpu.sync_copy(zeros_hbm, rows_v)
        for i in range(ROWS_PER_TILE // B):
            pltpu.sync_copy(rows_v,
                            acc.at[pl.ds(s * ROWS_PER_TILE + i * B, B)])
        plsc.subcore_barrier()

        def body(j, carry):
            pltpu.async_copy(y_hbm.at[src_v.at[j]], rows_v, sem).wait()
            pltpu.sync_copy(rows_v, acc.at[dst_v.at[j]], add=True)
            return carry

        lax.fori_loop(0, k, body, 0)
        plsc.subcore_barrier()
        pltpu.sync_copy(acc.at[pl.ds(s * ROWS_PER_TILE, ROWS_PER_TILE)],
                        out_hbm.at[c, pl.ds(s * ROWS_PER_TILE, ROWS_PER_TILE)])

    return pass_kernel


def _dis_block(deg_ref):
    d = deg_ref[0] + deg_ref[1] + 1.0
    return lax.rsqrt(d)


def _pre_body(x_ref, w1_ref, wr_ref, br_ref, deg_ref, y1_ref, res_ref):
    dis = _dis_block(deg_ref)
    xb = x_ref[...]
    y1_ref[...] = jnp.dot(xb, w1_ref[...],
                          preferred_element_type=jnp.float32) * dis
    res_ref[...] = jnp.dot(xb, wr_ref[...],
                           preferred_element_type=jnp.float32) + br_ref[...]


def _mid_body(acc_ref, y1_ref, b1_ref, w2_ref, deg_ref, y2_ref):
    dis = _dis_block(deg_ref)
    h = dis * (acc_ref[0] + acc_ref[1] + y1_ref[...]) + b1_ref[...]
    h = jnp.maximum(h, 0.0)
    y2_ref[...] = jnp.dot(h, w2_ref[...],
                          preferred_element_type=jnp.float32) * dis


def _post_body(acc_ref, y2_ref, b2_ref, res_ref, deg_ref, out_ref):
    dis = _dis_block(deg_ref)
    out_ref[...] = (dis * (acc_ref[0] + acc_ref[1] + y2_ref[...])
                    + b2_ref[...] + res_ref[...])


_row_spec = pl.BlockSpec((R, D), lambda i: (i, 0))
_acc_spec = pl.BlockSpec((NC, R, D), lambda i: (0, i, 0))
_deg_spec = pl.BlockSpec((NC, R, 1), lambda i: (0, i, 0))
_w_spec = pl.BlockSpec((D, D), lambda i: (0, 0))
_b_spec = pl.BlockSpec((1, D), lambda i: (0, 0))
_GRID = N_PAD // R


def _tc_pre(x_pad, W1, Wr, br2, degp):
    return pl.pallas_call(
        _pre_body,
        grid=(_GRID,),
        in_specs=[_row_spec, _w_spec, _w_spec, _b_spec, _deg_spec],
        out_specs=[_row_spec, _row_spec],
        out_shape=[jax.ShapeDtypeStruct((N_PAD, D), jnp.float32),
                   jax.ShapeDtypeStruct((N_PAD, D), jnp.float32)],
    )(x_pad, W1, Wr, br2, degp)


def _tc_mid(acc1, y1, b12, W2, degp):
    return pl.pallas_call(
        _mid_body,
        grid=(_GRID,),
        in_specs=[_acc_spec, _row_spec, _b_spec, _w_spec, _deg_spec],
        out_specs=_row_spec,
        out_shape=jax.ShapeDtypeStruct((N_PAD, D), jnp.float32),
    )(acc1, y1, b12, W2, degp)


def _tc_post(acc2, y2, b22, res, degp):
    return pl.pallas_call(
        _post_body,
        grid=(_GRID,),
        in_specs=[_acc_spec, _row_spec, _b_spec, _row_spec, _deg_spec],
        out_specs=_row_spec,
        out_shape=jax.ShapeDtypeStruct((N_PAD, D), jnp.float32),
    )(acc2, y2, b22, res, degp)


def kernel(x, edge_index, W1, b1, W2, b2, Wr, br):
    E = edge_index.shape[1]
    k = -(-E // (NW * B))
    e_pad = NW * B * k

    src = edge_index[0].astype(jnp.int32)
    dst = edge_index[1].astype(jnp.int32)
    src3 = jnp.concatenate(
        [src, jnp.zeros((e_pad - E,), jnp.int32)]).reshape(NW, k, B)
    dst3 = jnp.concatenate(
        [dst, jnp.full((e_pad - E,), N, jnp.int32)]).reshape(NW, k, B)

    x_pad = jnp.pad(x, ((0, N_PAD - N), (0, 0)))
    consts = jnp.concatenate([jnp.zeros((ROWS_PER_TILE,), jnp.float32),
                              jnp.ones((B,), jnp.float32)])
    zeros_rows = jnp.zeros((B, D), jnp.float32)
    br2 = br.reshape(1, D)
    b12 = b1.reshape(1, D)
    b22 = b2.reshape(1, D)

    deg_k = _make_deg_kernel(k)
    pass_k = _make_pass_kernel(k)

    degp = deg_k(dst3, consts).reshape(NC, N_PAD, 1)
    y1, res = _tc_pre(x_pad, W1, Wr, br2, degp)
    acc1 = pass_k(y1, src3, dst3, zeros_rows)
    y2 = _tc_mid(acc1, y1, b12, W2, degp)
    acc2 = pass_k(y2, src3, dst3, zeros_rows)
    out = _tc_post(acc2, y2, b22, res, degp)
    return out[:N]
